# Initial kernel scaffold; baseline (speedup 1.0000x reference)
import jax
import jax.numpy as jnp
from jax.experimental import pallas as pl


def kernel(adjacency_list, feature_vectors, W1, a_src1, a_dst1, b1, gn1_weight, gn1_bias, gn1_mean_scale, W2, a_src2, a_dst2, b2, gn2_weight, gn2_bias, gn2_mean_scale, W3, a_src3, a_dst3, b3):
    raise NotImplementedError("write your pallas kernel here")


# speedup vs baseline: 1.7436x; 1.7436x over previous
import functools

import jax
import jax.numpy as jnp
from jax.experimental import pallas as pl

N_NODES = 50000
BLK = 5000
NB = N_NODES // BLK
NEG_SLOPE = 0.2
EPS_DENOM = 1e-16
GN_EPS = 1e-5
PACK = 128


def _leaky(x):
    return jnp.where(x >= 0, x, NEG_SLOPE * x)


def _elu(x):
    return jnp.where(x > 0, x, jnp.exp(jnp.minimum(x, 0.0)) - 1.0)


def _transform_block(x, w_ref, asrc_ref, adst_ref, hsd_ref, mx_ref):
    c = w_ref.shape[1]
    h = jnp.dot(x, w_ref[...], preferred_element_type=jnp.float32)
    s = jnp.dot(h, asrc_ref[...], preferred_element_type=jnp.float32)
    d = jnp.dot(h, adst_ref[...], preferred_element_type=jnp.float32)
    pad = jnp.zeros((x.shape[0], PACK - c - 2), jnp.float32)
    hsd_ref[...] = jnp.concatenate([h, s, d, pad], axis=1)
    bm = jnp.concatenate([jnp.max(s)[None, None], jnp.max(d)[None, None]],
                         axis=1)

    @pl.when(pl.program_id(0) == 0)
    def _():
        mx_ref[...] = bm

    @pl.when(pl.program_id(0) != 0)
    def _():
        mx_ref[...] = jnp.maximum(mx_ref[...], bm)


def _first_kernel(x_ref, w_ref, asrc_ref, adst_ref, hsd_ref, mx_ref):
    _transform_block(x_ref[...], w_ref, asrc_ref, adst_ref, hsd_ref, mx_ref)


def _stats_kernel(seg_ref, bias_ref, x0_ref, m_ref):
    cp = bias_ref.shape[1]
    seg = seg_ref[...]
    x0 = seg[:, 1:cp + 1] / (seg[:, 0:1] + EPS_DENOM) + bias_ref[...]
    x0_ref[...] = x0
    bm = jnp.concatenate([jnp.sum(x0, axis=0, keepdims=True),
                          jnp.sum(x0 * x0, axis=0, keepdims=True)], axis=1)

    @pl.when(pl.program_id(0) == 0)
    def _():
        m_ref[...] = bm

    @pl.when(pl.program_id(0) != 0)
    def _():
        m_ref[...] = m_ref[...] + bm


def _norm_kernel(x0_ref, m_ref, gnw_ref, gnb_ref, gnm_ref,
                 w_ref, asrc_ref, adst_ref, hsd_ref, mx_ref):
    cp = gnw_ref.shape[1]
    m1 = m_ref[0:1, 0:cp] / N_NODES
    m2 = m_ref[0:1, cp:2 * cp] / N_NODES
    ms = gnm_ref[...]
    var = m2 - m1 * m1 * ms * (2.0 - ms)
    cen = x0_ref[...] - m1 * ms
    xn = gnw_ref[...] * cen / jnp.sqrt(var + GN_EPS) + gnb_ref[...]
    _transform_block(_elu(xn), w_ref, asrc_ref, adst_ref, hsd_ref, mx_ref)


def _final_kernel(seg_ref, bias_ref, out_ref):
    cp = bias_ref.shape[1]
    seg = seg_ref[...]
    out_ref[...] = seg[:, 1:cp + 1] / (seg[:, 0:1] + EPS_DENOM) + bias_ref[...]


def _row_spec(cols):
    return pl.BlockSpec((BLK, cols), lambda i: (i, 0))


def _full_spec(shape):
    return pl.BlockSpec(shape, lambda i: (0,) * len(shape))


def _dense_first(x, w, a_src, a_dst):
    c = w.shape[1]
    return pl.pallas_call(
        _first_kernel,
        grid=(NB,),
        in_specs=[_row_spec(x.shape[1]), _full_spec(w.shape),
                  _full_spec((c, 1)), _full_spec((c, 1))],
        out_specs=(_row_spec(PACK), _full_spec((1, 2))),
        out_shape=(
            jax.ShapeDtypeStruct((N_NODES, PACK), jnp.float32),
            jax.ShapeDtypeStruct((1, 2), jnp.float32),
        ),
    )(x, w, a_src.reshape(c, 1), a_dst.reshape(c, 1))


def _stats(seg, bias):
    cp = bias.shape[0]
    return pl.pallas_call(
        _stats_kernel,
        grid=(NB,),
        in_specs=[_row_spec(cp + 1), _full_spec((1, cp))],
        out_specs=(_row_spec(cp), _full_spec((1, 2 * cp))),
        out_shape=(
            jax.ShapeDtypeStruct((N_NODES, cp), jnp.float32),
            jax.ShapeDtypeStruct((1, 2 * cp), jnp.float32),
        ),
    )(seg, bias.reshape(1, cp))


def _norm_transform(x0, m, gnw, gnb, gnm, w, a_src, a_dst):
    cp = x0.shape[1]
    c = w.shape[1]
    return pl.pallas_call(
        _norm_kernel,
        grid=(NB,),
        in_specs=[_row_spec(cp), _full_spec((1, 2 * cp)),
                  _full_spec((1, cp)), _full_spec((1, cp)),
                  _full_spec((1, cp)), _full_spec(w.shape),
                  _full_spec((c, 1)), _full_spec((c, 1))],
        out_specs=(_row_spec(PACK), _full_spec((1, 2))),
        out_shape=(
            jax.ShapeDtypeStruct((N_NODES, PACK), jnp.float32),
            jax.ShapeDtypeStruct((1, 2), jnp.float32),
        ),
    )(x0, m, gnw.reshape(1, cp), gnb.reshape(1, cp), gnm.reshape(1, cp),
      w, a_src.reshape(c, 1), a_dst.reshape(c, 1))


def _dense_final(seg, bias):
    cp = bias.shape[0]
    return pl.pallas_call(
        _final_kernel,
        grid=(NB,),
        in_specs=[_row_spec(cp + 1), _full_spec((1, cp))],
        out_specs=_row_spec(cp),
        out_shape=jax.ShapeDtypeStruct((N_NODES, cp), jnp.float32),
    )(seg, bias.reshape(1, cp))


def _edge_aggregate(hsd, c_dim, mx, src, dst):
    cbound = _leaky(mx[0, 0] + mx[0, 1])
    e = jnp.take(hsd[:, c_dim], src) + jnp.take(hsd[:, c_dim + 1], dst)
    ex = jnp.exp(_leaky(e) - cbound)
    payload = jnp.concatenate(
        [ex[:, None], ex[:, None] * jnp.take(hsd[:, :c_dim], src, axis=0)],
        axis=1)
    return jax.ops.segment_sum(payload, dst, num_segments=N_NODES)


@jax.jit
def kernel(adjacency_list, feature_vectors,
           W1, a_src1, a_dst1, b1, gn1_weight, gn1_bias, gn1_mean_scale,
           W2, a_src2, a_dst2, b2, gn2_weight, gn2_bias, gn2_mean_scale,
           W3, a_src3, a_dst3, b3):
    src = adjacency_list[0]
    dst = adjacency_list[1]
    d1, d2, d3 = W1.shape[1], W2.shape[1], W3.shape[1]

    hsd1, mx1 = _dense_first(feature_vectors, W1, a_src1, a_dst1)
    seg1 = _edge_aggregate(hsd1, d1, mx1, src, dst)

    x01, m1 = _stats(seg1, b1)
    hsd2, mx2 = _norm_transform(x01, m1, gn1_weight, gn1_bias,
                                gn1_mean_scale, W2, a_src2, a_dst2)
    seg2 = _edge_aggregate(hsd2, d2, mx2, src, dst)

    x02, m2 = _stats(seg2, b2)
    hsd3, mx3 = _norm_transform(x02, m2, gn2_weight, gn2_bias,
                                gn2_mean_scale, W3, a_src3, a_dst3)
    seg3 = _edge_aggregate(hsd3, d3, mx3, src, dst)

    return _dense_final(seg3, b3)


# speedup vs baseline: 1.7785x; 1.0200x over previous
import functools

import jax
import jax.numpy as jnp
from jax.experimental import pallas as pl

N_NODES = 50000
BLK = 5000
NB = N_NODES // BLK
NEG_SLOPE = 0.2
EPS_DENOM = 1e-16
GN_EPS = 1e-5
PACK = 128


def _leaky(x):
    return jnp.where(x >= 0, x, NEG_SLOPE * x)


def _elu(x):
    return jnp.where(x > 0, x, jnp.exp(jnp.minimum(x, 0.0)) - 1.0)


def _transform_block(x, w_ref, asrc_ref, adst_ref, hsd_ref, mx_ref):
    c = w_ref.shape[1]
    h = jnp.dot(x, w_ref[...], preferred_element_type=jnp.float32)
    s = jnp.dot(h, asrc_ref[...], preferred_element_type=jnp.float32)
    d = jnp.dot(h, adst_ref[...], preferred_element_type=jnp.float32)
    pad = jnp.zeros((x.shape[0], PACK - c - 2), jnp.float32)
    hsd_ref[...] = jnp.concatenate([h, s, d, pad], axis=1)
    bm = jnp.concatenate([jnp.max(s)[None, None], jnp.max(d)[None, None]],
                         axis=1)

    @pl.when(pl.program_id(0) == 0)
    def _():
        mx_ref[...] = bm

    @pl.when(pl.program_id(0) != 0)
    def _():
        mx_ref[...] = jnp.maximum(mx_ref[...], bm)


def _first_kernel(x_ref, w_ref, asrc_ref, adst_ref, hsd_ref, mx_ref):
    _transform_block(x_ref[...], w_ref, asrc_ref, adst_ref, hsd_ref, mx_ref)


def _stats_kernel(seg_ref, bias_ref, x0_ref, m_ref):
    cp = bias_ref.shape[1]
    seg = seg_ref[...]
    x0 = seg[:, 1:cp + 1] / (seg[:, 0:1] + EPS_DENOM) + bias_ref[...]
    x0_ref[...] = x0
    bm = jnp.concatenate([jnp.sum(x0, axis=0, keepdims=True),
                          jnp.sum(x0 * x0, axis=0, keepdims=True)], axis=1)

    @pl.when(pl.program_id(0) == 0)
    def _():
        m_ref[...] = bm

    @pl.when(pl.program_id(0) != 0)
    def _():
        m_ref[...] = m_ref[...] + bm


def _norm_kernel(x0_ref, m_ref, gnw_ref, gnb_ref, gnm_ref,
                 w_ref, asrc_ref, adst_ref, hsd_ref, mx_ref):
    cp = gnw_ref.shape[1]
    m1 = m_ref[0:1, 0:cp] / N_NODES
    m2 = m_ref[0:1, cp:2 * cp] / N_NODES
    ms = gnm_ref[...]
    var = m2 - m1 * m1 * ms * (2.0 - ms)
    cen = x0_ref[...] - m1 * ms
    xn = gnw_ref[...] * cen / jnp.sqrt(var + GN_EPS) + gnb_ref[...]
    _transform_block(_elu(xn), w_ref, asrc_ref, adst_ref, hsd_ref, mx_ref)


def _final_kernel(seg_ref, bias_ref, out_ref):
    cp = bias_ref.shape[1]
    seg = seg_ref[...]
    out_ref[...] = seg[:, 1:cp + 1] / (seg[:, 0:1] + EPS_DENOM) + bias_ref[...]


def _row_spec(cols):
    return pl.BlockSpec((BLK, cols), lambda i: (i, 0))


def _full_spec(shape):
    return pl.BlockSpec(shape, lambda i: (0,) * len(shape))


def _dense_first(x, w, a_src, a_dst):
    c = w.shape[1]
    return pl.pallas_call(
        _first_kernel,
        grid=(NB,),
        in_specs=[_row_spec(x.shape[1]), _full_spec(w.shape),
                  _full_spec((c, 1)), _full_spec((c, 1))],
        out_specs=(_row_spec(PACK), _full_spec((1, 2))),
        out_shape=(
            jax.ShapeDtypeStruct((N_NODES, PACK), jnp.float32),
            jax.ShapeDtypeStruct((1, 2), jnp.float32),
        ),
    )(x, w, a_src.reshape(c, 1), a_dst.reshape(c, 1))


def _stats(seg, bias):
    cp = bias.shape[0]
    return pl.pallas_call(
        _stats_kernel,
        grid=(NB,),
        in_specs=[_row_spec(cp + 1), _full_spec((1, cp))],
        out_specs=(_row_spec(cp), _full_spec((1, 2 * cp))),
        out_shape=(
            jax.ShapeDtypeStruct((N_NODES, cp), jnp.float32),
            jax.ShapeDtypeStruct((1, 2 * cp), jnp.float32),
        ),
    )(seg, bias.reshape(1, cp))


def _norm_transform(x0, m, gnw, gnb, gnm, w, a_src, a_dst):
    cp = x0.shape[1]
    c = w.shape[1]
    return pl.pallas_call(
        _norm_kernel,
        grid=(NB,),
        in_specs=[_row_spec(cp), _full_spec((1, 2 * cp)),
                  _full_spec((1, cp)), _full_spec((1, cp)),
                  _full_spec((1, cp)), _full_spec(w.shape),
                  _full_spec((c, 1)), _full_spec((c, 1))],
        out_specs=(_row_spec(PACK), _full_spec((1, 2))),
        out_shape=(
            jax.ShapeDtypeStruct((N_NODES, PACK), jnp.float32),
            jax.ShapeDtypeStruct((1, 2), jnp.float32),
        ),
    )(x0, m, gnw.reshape(1, cp), gnb.reshape(1, cp), gnm.reshape(1, cp),
      w, a_src.reshape(c, 1), a_dst.reshape(c, 1))


def _dense_final(seg, bias):
    cp = bias.shape[0]
    return pl.pallas_call(
        _final_kernel,
        grid=(NB,),
        in_specs=[_row_spec(cp + 1), _full_spec((1, cp))],
        out_specs=_row_spec(cp),
        out_shape=jax.ShapeDtypeStruct((N_NODES, cp), jnp.float32),
    )(seg, bias.reshape(1, cp))


def _edge_aggregate(hsd, c_dim, mx, src, dst):
    cbound = _leaky(mx[0, 0] + mx[0, 1])
    e = jnp.take(hsd[:, c_dim], src) + jnp.take(hsd[:, c_dim + 1], dst)
    ex = jnp.exp(_leaky(e) - cbound)
    payload = jnp.concatenate(
        [ex[:, None], ex[:, None] * jnp.take(hsd[:, :c_dim], src, axis=0)],
        axis=1)
    return jax.ops.segment_sum(payload, dst, num_segments=N_NODES,
                               indices_are_sorted=True)


@jax.jit
def kernel(adjacency_list, feature_vectors,
           W1, a_src1, a_dst1, b1, gn1_weight, gn1_bias, gn1_mean_scale,
           W2, a_src2, a_dst2, b2, gn2_weight, gn2_bias, gn2_mean_scale,
           W3, a_src3, a_dst3, b3):
    perm = jnp.argsort(adjacency_list[1])
    src = jnp.take(adjacency_list[0], perm)
    dst = jnp.take(adjacency_list[1], perm)
    d1, d2, d3 = W1.shape[1], W2.shape[1], W3.shape[1]

    hsd1, mx1 = _dense_first(feature_vectors, W1, a_src1, a_dst1)
    seg1 = _edge_aggregate(hsd1, d1, mx1, src, dst)

    x01, m1 = _stats(seg1, b1)
    hsd2, mx2 = _norm_transform(x01, m1, gn1_weight, gn1_bias,
                                gn1_mean_scale, W2, a_src2, a_dst2)
    seg2 = _edge_aggregate(hsd2, d2, mx2, src, dst)

    x02, m2 = _stats(seg2, b2)
    hsd3, mx3 = _norm_transform(x02, m2, gn2_weight, gn2_bias,
                                gn2_mean_scale, W3, a_src3, a_dst3)
    seg3 = _edge_aggregate(hsd3, d3, mx3, src, dst)

    return _dense_final(seg3, b3)
